# baseline (speedup 1.0000x reference)
import jax
import jax.numpy as jnp
from jax import lax
from jax.experimental import pallas as pl
from jax.experimental.pallas import tpu as pltpu
from jax.experimental.pallas import tpu_sc as plsc

B = 1024
L = 50
CLIP = 256
HIDDEN = 32
PAIRS = L * L
NLANE = 16
NSTEP = 160
PAD_PAIRS = NSTEP * NLANE
NW = 32
B_PER_W = B // NW
LPAD = 64
SENT = 2147483647
CHUNK = 128
NCHUNK = 20
NHALF = 2
HROWS = PAD_PAIRS // NHALF
CPH = NCHUNK // NHALF
HSTEPS = HROWS // NLANE


def _sc_body(ts_hbm, emb_hbm, out_hbm, ts_v, d_v, table_v, pos_v,
             buf0, buf1, shr, sem):
    sid = lax.axis_index("s")
    wid = sid * 2 + lax.axis_index("c")
    bufs = (buf0, buf1)
    pltpu.sync_copy(emb_hbm, table_v)
    pltpu.sync_copy(ts_hbm.at[pl.ds(wid * B_PER_W, B_PER_W)], ts_v)

    lanes = lax.iota(jnp.int32, NLANE)
    lv = jnp.full((NLANE,), L, jnp.int32)
    pairsv = jnp.full((NLANE,), PAIRS, jnp.int32)
    lastv = jnp.full((NLANE,), PAIRS - 1, jnp.int32)
    zerov = jnp.full((NLANE,), 0, jnp.int32)
    onev = jnp.full((NLANE,), 1, jnp.int32)
    sentv = jnp.full((NLANE,), SENT, jnp.int32)
    clipv = jnp.full((NLANE,), CLIP, jnp.int32)
    clip1v = jnp.full((NLANE,), CLIP + 1, jnp.int32)
    onefv = jnp.full((NLANE,), 1.0, jnp.float32)

    def pinit(s, _):
        vals = jnp.minimum(lanes + jnp.full((NLANE,), s * NLANE, jnp.int32),
                           lastv)
        pos_v[s // 8, pl.ds((s % 8) * NLANE, NLANE)] = vals
        return jnp.int32(0)

    lax.fori_loop(jnp.int32(0), jnp.int32(NSTEP), pinit, jnp.int32(0))

    def per_b(bi, carry):
        b = wid * B_PER_W + bi
        biv = jnp.full((NLANE,), bi, jnp.int32)

        def p1(s, macc):
            k = lanes + jnp.full((NLANE,), s * NLANE, jnp.int32)
            kc = jnp.minimum(k, lastv)
            i = lax.div(kc, lv)
            j = kc - i * lv
            ti = plsc.load_gather(ts_v, [biv, i])
            tj = plsc.load_gather(ts_v, [biv, j])
            diff = ti - tj
            d = jnp.maximum(diff, zerov - diff)
            q = jnp.where((k < pairsv) & (d != zerov), d, sentv)
            d_v[pl.ds(s * NLANE, NLANE)] = d
            return jnp.minimum(macc, q)

        macc = lax.fori_loop(
            jnp.int32(0), jnp.int32(NSTEP), p1,
            jnp.full((NLANE,), SENT, jnp.int32),
        )
        tmin = jnp.min(macc)
        tminv = jnp.full((NLANE,), tmin, jnp.int32)
        rcpv = onefv / tminv.astype(jnp.float32)

        for h in range(NHALF):
            buf = bufs[h]


            @plsc.parallel_loop(jnp.int32(0), jnp.int32(HSTEPS),
                                step=jnp.int32(1))
            def p2(t, h=h, buf=buf):
                d = d_v[pl.ds(jnp.int32(h * HROWS) + t * NLANE, NLANE)]
                qi = (d.astype(jnp.float32) * rcpv).astype(jnp.int32)
                qc = jnp.minimum(qi, clip1v)
                qc = qc + jnp.where((qc + onev) * tminv <= d, onev, zerov)
                qc = qc - jnp.where(qc * tminv > d, onev, zerov)
                q = jnp.minimum(qc, clipv)
                base = t * NLANE
                for r in range(NLANE):
                    sidx = q[r]
                    row = base + r
                    buf[row, pl.ds(0, NLANE)] = table_v[sidx, pl.ds(0, NLANE)]
                    buf[row, pl.ds(NLANE, NLANE)] = (
                        table_v[sidx, pl.ds(NLANE, NLANE)])
            pltpu.sync_copy(buf.at[pl.ds(0, HROWS // 2)], shr.at[sid])
        return carry

    lax.fori_loop(jnp.int32(0), jnp.int32(B_PER_W), per_b, jnp.int32(0))


@jax.jit
def _run(ts_pad, time_emb):
    mesh = plsc.VectorSubcoreMesh(core_axis_name="c", subcore_axis_name="s")
    f = pl.kernel(
        _sc_body,
        out_type=jax.ShapeDtypeStruct((B, PAIRS, HIDDEN), jnp.float32),
        mesh=mesh,
        scratch_types=[
            pltpu.VMEM((B_PER_W, LPAD), jnp.int32),
            pltpu.VMEM((PAD_PAIRS,), jnp.int32),
            pltpu.VMEM((CLIP + 1, HIDDEN), jnp.float32),
            pltpu.VMEM((NCHUNK, CHUNK), jnp.int32),
            pltpu.VMEM((HROWS, HIDDEN), jnp.float32),
            pltpu.VMEM((HROWS, HIDDEN), jnp.float32),
            pltpu.VMEM_SHARED((16, HROWS // 2, HIDDEN), jnp.float32),
            pltpu.SemaphoreType.DMA,
        ],
        compiler_params=pltpu.CompilerParams(
            needs_layout_passes=False, use_tc_tiling_on_sc=False,
        ),
    )
    return f(ts_pad, time_emb)


def kernel(timestamps, time_emb):
    ts32 = timestamps.astype(jnp.int32)
    ts_pad = jnp.zeros((B, LPAD), jnp.int32).at[:, :L].set(ts32)
    out = _run(ts_pad, time_emb.astype(jnp.float32))
    return out.reshape(B, L, L, HIDDEN)


# speedup vs baseline: 1.0286x; 1.0286x over previous
import jax
import jax.numpy as jnp
from jax import lax
from jax.experimental import pallas as pl
from jax.experimental.pallas import tpu as pltpu
from jax.experimental.pallas import tpu_sc as plsc

B = 1024
L = 50
CLIP = 256
HIDDEN = 32
PAIRS = L * L
NLANE = 16
NSTEP = 160
PAD_PAIRS = NSTEP * NLANE
NW = 32
B_PER_W = B // NW
LPAD = 64
SENT = 2147483647
CHUNK = 128
NCHUNK = 20
NHALF = 2
HROWS = PAD_PAIRS // NHALF
CPH = NCHUNK // NHALF
HSTEPS = HROWS // NLANE


def _sc_body(ts_hbm, emb_hbm, out_hbm, ts_v, d_v, table_v, pos_v,
             buf0, buf1, sem):
    wid = lax.axis_index("s") * 2 + lax.axis_index("c")
    bufs = (buf0, buf1)
    pltpu.sync_copy(emb_hbm, table_v)
    pltpu.sync_copy(ts_hbm.at[pl.ds(wid * B_PER_W, B_PER_W)], ts_v)

    lanes = lax.iota(jnp.int32, NLANE)
    lv = jnp.full((NLANE,), L, jnp.int32)
    pairsv = jnp.full((NLANE,), PAIRS, jnp.int32)
    lastv = jnp.full((NLANE,), PAIRS - 1, jnp.int32)
    zerov = jnp.full((NLANE,), 0, jnp.int32)
    onev = jnp.full((NLANE,), 1, jnp.int32)
    sentv = jnp.full((NLANE,), SENT, jnp.int32)
    clipv = jnp.full((NLANE,), CLIP, jnp.int32)
    clip1v = jnp.full((NLANE,), CLIP + 1, jnp.int32)
    onefv = jnp.full((NLANE,), 1.0, jnp.float32)

    def pinit(s, _):
        vals = jnp.minimum(lanes + jnp.full((NLANE,), s * NLANE, jnp.int32),
                           lastv)
        pos_v[s // 8, pl.ds((s % 8) * NLANE, NLANE)] = vals
        return jnp.int32(0)

    lax.fori_loop(jnp.int32(0), jnp.int32(NSTEP), pinit, jnp.int32(0))

    def per_b(bi, carry):
        b = wid * B_PER_W + bi
        biv = jnp.full((NLANE,), bi, jnp.int32)

        def p1(s, macc):
            k = lanes + jnp.full((NLANE,), s * NLANE, jnp.int32)
            kc = jnp.minimum(k, lastv)
            i = lax.div(kc, lv)
            j = kc - i * lv
            ti = plsc.load_gather(ts_v, [biv, i])
            tj = plsc.load_gather(ts_v, [biv, j])
            diff = ti - tj
            d = jnp.maximum(diff, zerov - diff)
            q = jnp.where((k < pairsv) & (d != zerov), d, sentv)
            d_v[pl.ds(s * NLANE, NLANE)] = d
            return jnp.minimum(macc, q)

        macc = lax.fori_loop(
            jnp.int32(0), jnp.int32(NSTEP), p1,
            jnp.full((NLANE,), SENT, jnp.int32),
        )
        tmin = jnp.min(macc)
        tminv = jnp.full((NLANE,), tmin, jnp.int32)
        rcpv = onefv / tminv.astype(jnp.float32)

        for h in range(NHALF):
            buf = bufs[h]

            @pl.when(bi > 0)
            def _drain(buf=buf):
                pltpu.make_async_copy(
                    buf, out_hbm.at[b, pl.ds(jnp.int32(0), HROWS)], sem,
                ).wait()

            @plsc.parallel_loop(jnp.int32(0), jnp.int32(HSTEPS),
                                step=jnp.int32(1))
            def p2(t, h=h, buf=buf):
                d = d_v[pl.ds(jnp.int32(h * HROWS) + t * NLANE, NLANE)]
                qi = (d.astype(jnp.float32) * rcpv).astype(jnp.int32)
                qc = jnp.minimum(qi, clip1v)
                qc = qc + jnp.where((qc + onev) * tminv <= d, onev, zerov)
                qc = qc - jnp.where(qc * tminv > d, onev, zerov)
                q = jnp.minimum(qc, clipv)
                base = t * NLANE
                for r in range(NLANE):
                    sidx = q[r]
                    row = base + r
                    buf[row, pl.ds(0, NLANE)] = table_v[sidx, pl.ds(0, NLANE)]
                    buf[row, pl.ds(NLANE, NLANE)] = (
                        table_v[sidx, pl.ds(NLANE, NLANE)])
            for j in range(CPH):
                pltpu.async_copy(
                    buf.at[pl.ds(jnp.int32(j * CHUNK), CHUNK)],
                    out_hbm.at[b].at[pos_v.at[jnp.int32(h * CPH + j)]],
                    sem,
                )
        return carry

    lax.fori_loop(jnp.int32(0), jnp.int32(B_PER_W), per_b, jnp.int32(0))

    for h in range(NHALF):
        pltpu.make_async_copy(
            bufs[h], out_hbm.at[jnp.int32(0), pl.ds(jnp.int32(0), HROWS)],
            sem,
        ).wait()


@jax.jit
def _run(ts_pad, time_emb):
    mesh = plsc.VectorSubcoreMesh(core_axis_name="c", subcore_axis_name="s")
    f = pl.kernel(
        _sc_body,
        out_type=jax.ShapeDtypeStruct((B, PAIRS, HIDDEN), jnp.float32),
        mesh=mesh,
        scratch_types=[
            pltpu.VMEM((B_PER_W, LPAD), jnp.int32),
            pltpu.VMEM((PAD_PAIRS,), jnp.int32),
            pltpu.VMEM((CLIP + 1, HIDDEN), jnp.float32),
            pltpu.VMEM((NCHUNK, CHUNK), jnp.int32),
            pltpu.VMEM((HROWS, HIDDEN), jnp.float32),
            pltpu.VMEM((HROWS, HIDDEN), jnp.float32),
            pltpu.SemaphoreType.DMA,
        ],
        compiler_params=pltpu.CompilerParams(
            needs_layout_passes=False, use_tc_tiling_on_sc=False,
        ),
    )
    return f(ts_pad, time_emb)


def kernel(timestamps, time_emb):
    ts32 = timestamps.astype(jnp.int32)
    ts_pad = jnp.zeros((B, LPAD), jnp.int32).at[:, :L].set(ts32)
    out = _run(ts_pad, time_emb.astype(jnp.float32))
    return out.reshape(B, L, L, HIDDEN)
